# Initial kernel scaffold; baseline (speedup 1.0000x reference)
import jax
import jax.numpy as jnp
from jax.experimental import pallas as pl


def kernel(x, emb):
    raise NotImplementedError("write your pallas kernel here")


# speedup vs baseline: 1.3573x; 1.3573x over previous
import functools

import jax
import jax.numpy as jnp
from jax.experimental import pallas as pl
from jax.experimental.pallas import tpu as pltpu

_M = 1024
_D = 256
_BN = 4608
_BLK = 512
_GRID = _BN // _BLK


def _vq_body(x_ref, emb_ref, z_ref, idx_ref, loss_ref, perp_ref,
             counts_scr, sse_scr):
    i = pl.program_id(0)

    @pl.when(i == 0)
    def _init():
        counts_scr[...] = jnp.zeros_like(counts_scr)
        sse_scr[0, 0] = 0.0

    xb = x_ref[...]
    emb = emb_ref[...]

    sim = jax.lax.dot_general(
        xb, emb, (((1,), (1,)), ((), ())),
        preferred_element_type=jnp.float32)
    l2q = jnp.sum(xb * xb, axis=1, keepdims=True)
    l2k = jnp.sum(emb * emb, axis=1).reshape(1, _M)
    neg_dist = -((l2q + l2k) - 2.0 * sim)

    mx = jnp.max(neg_dist, axis=1, keepdims=True)
    iot = jax.lax.broadcasted_iota(jnp.int32, (_BLK, _M), 1)
    idx = jnp.min(jnp.where(neg_dist == mx, iot, _M),
                  axis=1, keepdims=True)
    idx_ref[...] = idx

    onehot = (iot == idx).astype(jnp.float32)
    z_ref[...] = jax.lax.dot_general(
        onehot, emb, (((1,), (0,)), ((), ())),
        preferred_element_type=jnp.float32)

    counts_scr[...] += jnp.sum(onehot, axis=0, keepdims=True)
    sse_scr[0, 0] += -jnp.sum(mx)

    @pl.when(i == _GRID - 1)
    def _finish():
        ones11 = jnp.ones((1, 1), jnp.float32)
        loss_ref[...] = (sse_scr[0, 0] / float(_BN * _D)) * ones11
        mean = counts_scr[...] * (1.0 / float(_BN))
        perp_ref[...] = jnp.exp(-jnp.sum(mean * jnp.log(mean + 1e-10))) * ones11


@functools.partial(jax.jit)
def kernel(x, emb):
    b, n, d = x.shape
    q = x.reshape(b * n, d)

    z, idx, loss, perp = pl.pallas_call(
        _vq_body,
        grid=(_GRID,),
        in_specs=[
            pl.BlockSpec((_BLK, _D), lambda i: (i, 0)),
            pl.BlockSpec((_M, _D), lambda i: (0, 0)),
        ],
        out_specs=[
            pl.BlockSpec((_BLK, _D), lambda i: (i, 0)),
            pl.BlockSpec((_BLK, 1), lambda i: (i, 0)),
            pl.BlockSpec((1, 1), lambda i: (0, 0)),
            pl.BlockSpec((1, 1), lambda i: (0, 0)),
        ],
        out_shape=[
            jax.ShapeDtypeStruct((_BN, _D), jnp.float32),
            jax.ShapeDtypeStruct((_BN, 1), jnp.int32),
            jax.ShapeDtypeStruct((1, 1), jnp.float32),
            jax.ShapeDtypeStruct((1, 1), jnp.float32),
        ],
        scratch_shapes=[
            pltpu.VMEM((1, _M), jnp.float32),
            pltpu.SMEM((1, 1), jnp.float32),
        ],
        compiler_params=pltpu.CompilerParams(
            dimension_semantics=("arbitrary",)),
    )(q, emb)

    quantized = z.reshape(b, n, d)
    indices_out = idx.reshape(b, 1, n)
    return quantized, loss[0, 0], indices_out, perp.reshape(1)
